# baseline (speedup 1.0000x reference)
import functools

import jax
import jax.numpy as jnp
from jax import lax
from jax.experimental import pallas as pl
from jax.experimental.pallas import tpu as pltpu
from jax.experimental.pallas import tpu_sc as plsc

NC = 2
NS = 16
NW = NC * NS
L = 16


def _make_lookup(X0: int, X1: int, V: int, D: int):
  IT = X0 // 128
  JT = X1 // 8
  assert X0 % 128 == 0 and X1 % 8 == 0 and IT % NW == 0
  it_per_w = IT // NW
  mesh = plsc.VectorSubcoreMesh(
      core_axis_name="c", subcore_axis_name="s", num_cores=NC,
      num_subcores=NS)

  @functools.partial(
      pl.kernel,
      out_type=jax.ShapeDtypeStruct((D, JT, IT, 8, 128), jnp.float32),
      mesh=mesh,
      scratch_types=[
          pltpu.VMEM((128, X1), jnp.int32),
          pltpu.VMEM((D, 8, 128), jnp.float32),
          pltpu.VMEM((V * D,), jnp.float32),
          pltpu.SemaphoreType.DMA,
      ],
      compiler_params=pltpu.CompilerParams(
          use_tc_tiling_on_sc=False, needs_layout_passes=False),
  )
  def lookup(x_hbm, wte_hbm, out_hbm, xblk, out_t, tab_v, sem):
    wid = lax.axis_index("s") * NC + lax.axis_index("c")
    pltpu.sync_copy(wte_hbm, tab_v)
    iota = lax.iota(jnp.int32, L)
    zero = iota * 0

    def do_jt(jt, it):
      @plsc.parallel_loop(0, 64, unroll=4)
      def group(g):
        js = g // 8
        ig = g - js * 8
        jv = zero + (jt * 8 + js)
        i16 = ig * L + iota
        xg = plsc.load_gather(xblk, [i16, jv])
        src = xg * D
        for d in range(D):
          vals = plsc.load_gather(tab_v, [src + d])
          out_t[d, js, pl.ds(ig * L, L)] = vals
      pltpu.sync_copy(out_t, out_hbm.at[:, jt, it])
      return it

    def do_it(a, carry):
      it = wid * it_per_w + a
      pltpu.sync_copy(x_hbm.at[pl.ds(it * 128, 128)], xblk)
      lax.fori_loop(0, JT, do_jt, it)
      return carry

    lax.fori_loop(0, it_per_w, do_it, 0)

  return lookup


def kernel(x, wte):
  X0, X1 = x.shape
  V, D = wte.shape
  out5 = _make_lookup(X0, X1, V, D)(x, wte.reshape(V * D))
  return out5.transpose(2, 4, 1, 3, 0).reshape(X0, X1, D)


# speedup vs baseline: 1.8414x; 1.8414x over previous
import functools

import jax
import jax.numpy as jnp
from jax import lax
from jax.experimental import pallas as pl
from jax.experimental.pallas import tpu as pltpu
from jax.experimental.pallas import tpu_sc as plsc

NC = 2
NS = 16
NW = NC * NS
L = 16


def _make_lookup(X0: int, X1: int, V: int, D: int):
  IT = X0 // 128
  JT = X1 // 8
  assert X0 % 128 == 0 and X1 % 8 == 0 and IT % NW == 0
  it_per_w = IT // NW
  mesh = plsc.VectorSubcoreMesh(
      core_axis_name="c", subcore_axis_name="s", num_cores=NC,
      num_subcores=NS)

  @functools.partial(
      pl.kernel,
      out_type=jax.ShapeDtypeStruct((D, JT, IT, 8, 128), jnp.float32),
      mesh=mesh,
      scratch_types=[
          pltpu.VMEM((128, X1), jnp.int32),
          pltpu.VMEM((D, 8, 128), jnp.float32),
          pltpu.VMEM((V * D,), jnp.float32),
          pltpu.SemaphoreType.DMA,
      ],
      compiler_params=pltpu.CompilerParams(
          use_tc_tiling_on_sc=False, needs_layout_passes=False),
  )
  def lookup(x_hbm, wte_hbm, out_hbm, xblk, out_t, tab_v, sem):
    wid = lax.axis_index("s") * NC + lax.axis_index("c")
    pltpu.sync_copy(wte_hbm, tab_v)
    iota = lax.iota(jnp.int32, L)
    zero = iota * 0

    def do_jt(jt, it):
      @plsc.parallel_loop(0, 64, unroll=1)
      def group(g):
        js = g // 8
        ig = g - js * 8
        jv = zero + (jt * 8 + js)
        i16 = ig * L + iota
        xg = plsc.load_gather(xblk, [i16, jv])
        src = xg * D
        for d in range(D):
          vals = plsc.load_gather(tab_v, [src + d])
          out_t[d, js, pl.ds(ig * L, L)] = vals
      pltpu.sync_copy(out_t, out_hbm.at[:, jt, it])
      return it

    def do_it(a, carry):
      it = wid * it_per_w + a
      pltpu.sync_copy(x_hbm.at[pl.ds(it * 128, 128)], xblk)
      lax.fori_loop(0, JT, do_jt, it)
      return carry

    lax.fori_loop(0, it_per_w, do_it, 0)

  return lookup


def kernel(x, wte):
  X0, X1 = x.shape
  V, D = wte.shape
  out5 = _make_lookup(X0, X1, V, D)(x, wte.reshape(V * D))
  return out5.transpose(2, 4, 1, 3, 0).reshape(X0, X1, D)


# speedup vs baseline: 2.5501x; 1.3848x over previous
import functools

import jax
import jax.numpy as jnp
from jax import lax
from jax.experimental import pallas as pl
from jax.experimental.pallas import tpu as pltpu
from jax.experimental.pallas import tpu_sc as plsc

NC = 2
NS = 16
NW = NC * NS
L = 16


def _make_lookup(X0: int, X1: int, V: int, D: int):
  IT = X0 // 128
  JT = X1 // 8
  assert X0 % 128 == 0 and X1 % 8 == 0 and IT % NW == 0
  it_per_w = IT // NW
  n_units = it_per_w * JT
  assert n_units % 2 == 0
  mesh = plsc.VectorSubcoreMesh(
      core_axis_name="c", subcore_axis_name="s", num_cores=NC,
      num_subcores=NS)

  @functools.partial(
      pl.kernel,
      out_type=jax.ShapeDtypeStruct((D, JT, IT, 8, 128), jnp.float32),
      mesh=mesh,
      scratch_types=[
          pltpu.VMEM((128, X1), jnp.int32),
          pltpu.VMEM((2, D, 8, 128), jnp.float32),
          pltpu.VMEM((V * D,), jnp.float32),
          pltpu.SemaphoreType.DMA,
          pltpu.SemaphoreType.DMA,
      ],
      compiler_params=pltpu.CompilerParams(
          use_tc_tiling_on_sc=False, needs_layout_passes=False),
  )
  def lookup(x_hbm, wte_hbm, out_hbm, xblk, out_t2, tab_v, sem0, sem1):
    wid = lax.axis_index("s") * NC + lax.axis_index("c")
    pltpu.sync_copy(wte_hbm, tab_v)
    iota = lax.iota(jnp.int32, L)
    zero = iota * 0
    sems = (sem0, sem1)

    def unit(u, b):
      a = u // JT
      jt = u - a * JT
      it = wid * it_per_w + a

      @pl.when(jt == 0)
      def _():
        pltpu.sync_copy(x_hbm.at[pl.ds(it * 128, 128)], xblk)

      @pl.when(u >= 2)
      def _():
        pltpu.make_async_copy(
            out_t2.at[b], out_hbm.at[:, jt, it], sems[b]).wait()

      @plsc.parallel_loop(0, 64)
      def group(g):
        js = g // 8
        ig = g - js * 8
        jv = zero + (jt * 8 + js)
        i16 = ig * L + iota
        xg = plsc.load_gather(xblk, [i16, jv])
        src = xg * D
        for d in range(D):
          vals = plsc.load_gather(tab_v, [src + d])
          out_t2[b, d, js, pl.ds(ig * L, L)] = vals

      pltpu.async_copy(out_t2.at[b], out_hbm.at[:, jt, it], sems[b])

    def pair(u2, carry):
      unit(u2 * 2, 0)
      unit(u2 * 2 + 1, 1)
      return carry

    lax.fori_loop(0, n_units // 2, pair, 0)
    pltpu.make_async_copy(out_t2.at[0], out_hbm.at[:, 0, 0], sem0).wait()
    pltpu.make_async_copy(out_t2.at[1], out_hbm.at[:, 0, 0], sem1).wait()

  return lookup


def kernel(x, wte):
  X0, X1 = x.shape
  V, D = wte.shape
  out5 = _make_lookup(X0, X1, V, D)(x, wte.reshape(V * D))
  return out5.transpose(2, 4, 1, 3, 0).reshape(X0, X1, D)
